# baseline (speedup 1.0000x reference)
import functools

import jax
import jax.numpy as jnp
from jax.experimental import pallas as pl
from jax.experimental.pallas import tpu as pltpu

L = 17
B = 128
S = 512


def _viterbi_step(dp, atb_ref):
    cands = [dp[k:k + 1, :] + atb_ref[k] for k in range(L)]
    vals = cands
    while len(vals) > 1:
        nxt = [jnp.maximum(vals[i], vals[i + 1])
               for i in range(0, len(vals) - 1, 2)]
        if len(vals) % 2:
            nxt.append(vals[-1])
        vals = nxt
    best = vals[0]
    besti = jnp.zeros((L, B), jnp.int32)
    for k in range(L - 1, 0, -1):
        besti = jnp.where(cands[k] == best, k, besti)
    return best, besti


def _crf_kernel(xt_ref, yt_ref, A_ref, AT_ref, path_ref, nll_ref,
                bp_ref, atb_ref):
    A = A_ref[...]
    AT = AT_ref[...]
    E = jnp.exp(AT)

    for k in range(L):
        atb_ref[k] = jnp.broadcast_to(AT[:, k:k + 1], (L, B))

    lane_iota = jax.lax.broadcasted_iota(jnp.int32, (L, B), 0)

    x0 = xt_ref[0]
    y0 = yt_ref[pl.ds(0, 1), :]

    dp0 = x0
    m0 = jnp.max(x0, axis=0, keepdims=True)
    w0 = jax.lax.dot(E, jnp.exp(x0 - m0),
                     preferred_element_type=jnp.float32)
    acc0 = jnp.where(lane_iota == y0, x0, 0.0)

    def step(j, carry):
        dp, w, logacc, acc, yprev = carry
        xj = xt_ref[j]
        yj = yt_ref[pl.ds(j, 1), :]

        best, besti = _viterbi_step(dp, atb_ref)
        bp_ref[j] = besti
        dp_new = best + xj

        s = jnp.max(w, axis=0, keepdims=True)
        rs = 1.0 / s
        wn = jax.lax.dot(E, w * jnp.exp(xj),
                         preferred_element_type=jnp.float32) * rs
        logacc_n = logacc + jnp.log(s)

        ohprev = (lane_iota == yprev).astype(jnp.float32)
        acols = jax.lax.dot(A, ohprev,
                            preferred_element_type=jnp.float32)
        acc_n = acc + jnp.where(lane_iota == yj, xj + acols, 0.0)
        return dp_new, wn, logacc_n, acc_n, yj

    dp, w, logacc, acc, yprev = jax.lax.fori_loop(
        1, S - 1, step, (dp0, w0, m0, acc0, y0))

    xl = xt_ref[S - 1]
    yl = yt_ref[pl.ds(S - 1, 1), :]
    best, besti = _viterbi_step(dp, atb_ref)
    bp_ref[S - 1] = besti
    dp_last = best + xl

    alpha = logacc + jnp.log(w)
    v = xl + alpha
    mz = jnp.max(v, axis=0, keepdims=True)
    z = mz + jnp.log(jnp.sum(jnp.exp(v - mz), axis=0, keepdims=True))

    ohprev = (lane_iota == yprev).astype(jnp.float32)
    acols = jax.lax.dot(A, ohprev, preferred_element_type=jnp.float32)
    acc = acc + jnp.where(lane_iota == yl, xl + acols, 0.0)

    s = jnp.sum(acc, axis=0, keepdims=True)
    nll_ref[...] = jnp.sum(z - s, axis=1, keepdims=True) * (1.0 / B)

    last = jnp.zeros((1, B), jnp.int32)
    bestv = dp_last[0:1, :]
    for k in range(1, L):
        row = dp_last[k:k + 1, :]
        gt = row > bestv
        bestv = jnp.where(gt, row, bestv)
        last = jnp.where(gt, k, last)
    path_ref[pl.ds(S - 1, 1), :] = last

    def back(t, cur):
        j = S - 1 - t
        bprow = bp_ref[j]
        prev = jnp.max(jnp.where(lane_iota == cur, bprow, 0),
                       axis=0, keepdims=True)
        path_ref[pl.ds(j - 1, 1), :] = prev
        return prev

    jax.lax.fori_loop(0, S - 1, back, last)


@functools.partial(jax.jit, static_argnames=())
def kernel(x, y, A):
    xt = jnp.transpose(x, (1, 2, 0))
    yt = jnp.transpose(y, (1, 0))
    AT = jnp.transpose(A, (1, 0))

    path_t, nll = pl.pallas_call(
        _crf_kernel,
        out_shape=(
            jax.ShapeDtypeStruct((S, B), jnp.int32),
            jax.ShapeDtypeStruct((1, 1), jnp.float32),
        ),
        scratch_shapes=[
            pltpu.VMEM((S, L, B), jnp.int32),
            pltpu.VMEM((L, L, B), jnp.float32),
        ],
    )(xt, yt, A, AT)

    return path_t.T, nll[0, 0]


# speedup vs baseline: 1.0989x; 1.0989x over previous
import functools

import jax
import jax.numpy as jnp
from jax.experimental import pallas as pl
from jax.experimental.pallas import tpu as pltpu

L = 17
B = 128
S = 512


def _viterbi_step(dp, atb_ref):
    vals = [dp[k:k + 1, :] + atb_ref[k] for k in range(L)]
    idxs = list(range(L))
    while len(vals) > 1:
        nv, ni = [], []
        for i in range(0, len(vals) - 1, 2):
            a, b = vals[i], vals[i + 1]
            ia, ib = idxs[i], idxs[i + 1]
            if isinstance(ia, int):
                ia = jnp.full((L, B), ia, jnp.int32)
            if isinstance(ib, int):
                ib = jnp.full((L, B), ib, jnp.int32)
            gt = b > a
            nv.append(jnp.where(gt, b, a))
            ni.append(jnp.where(gt, ib, ia))
        if len(vals) % 2:
            nv.append(vals[-1])
            ni.append(idxs[-1])
        vals, idxs = nv, ni
    return vals[0], idxs[0]


def _crf_kernel(xt_ref, yt_ref, A_ref, AT_ref, path_ref, nll_ref,
                bp_ref, atb_ref):
    A = A_ref[...]
    AT = AT_ref[...]
    E = jnp.exp(AT)

    for k in range(L):
        atb_ref[k] = jnp.broadcast_to(AT[:, k:k + 1], (L, B))

    lane_iota = jax.lax.broadcasted_iota(jnp.int32, (L, B), 0)

    x0 = xt_ref[0]
    y0 = yt_ref[pl.ds(0, 1), :]

    dp0 = x0
    m0 = jnp.max(x0, axis=0, keepdims=True)
    w0 = jax.lax.dot(E, jnp.exp(x0 - m0),
                     preferred_element_type=jnp.float32)
    acc0 = jnp.where(lane_iota == y0, x0, 0.0)

    def step(j, carry):
        dp, w, logacc, acc, yprev = carry
        xj = xt_ref[j]
        yj = yt_ref[pl.ds(j, 1), :]

        best, besti = _viterbi_step(dp, atb_ref)
        bp_ref[j] = besti
        dp_new = best + xj

        s = jnp.max(w, axis=0, keepdims=True)
        rs = 1.0 / s
        wn = jax.lax.dot(E, w * jnp.exp(xj),
                         preferred_element_type=jnp.float32) * rs
        logacc_n = logacc + jnp.log(s)

        ohprev = (lane_iota == yprev).astype(jnp.float32)
        acols = jax.lax.dot(A, ohprev,
                            preferred_element_type=jnp.float32)
        acc_n = acc + jnp.where(lane_iota == yj, xj + acols, 0.0)
        return dp_new, wn, logacc_n, acc_n, yj

    def step2(t, carry):
        return step(2 * t + 2, step(2 * t + 1, carry))

    dp, w, logacc, acc, yprev = jax.lax.fori_loop(
        0, (S - 2) // 2, step2, (dp0, w0, m0, acc0, y0))

    xl = xt_ref[S - 1]
    yl = yt_ref[pl.ds(S - 1, 1), :]
    best, besti = _viterbi_step(dp, atb_ref)
    bp_ref[S - 1] = besti
    dp_last = best + xl

    alpha = logacc + jnp.log(w)
    v = xl + alpha
    mz = jnp.max(v, axis=0, keepdims=True)
    z = mz + jnp.log(jnp.sum(jnp.exp(v - mz), axis=0, keepdims=True))

    ohprev = (lane_iota == yprev).astype(jnp.float32)
    acols = jax.lax.dot(A, ohprev, preferred_element_type=jnp.float32)
    acc = acc + jnp.where(lane_iota == yl, xl + acols, 0.0)

    s = jnp.sum(acc, axis=0, keepdims=True)
    nll_ref[...] = jnp.sum(z - s, axis=1, keepdims=True) * (1.0 / B)

    last = jnp.zeros((1, B), jnp.int32)
    bestv = dp_last[0:1, :]
    for k in range(1, L):
        row = dp_last[k:k + 1, :]
        gt = row > bestv
        bestv = jnp.where(gt, row, bestv)
        last = jnp.where(gt, k, last)
    path_ref[pl.ds(S - 1, 1), :] = last

    def back(t, cur):
        j = S - 1 - t
        bprow = bp_ref[j]
        prev = jnp.max(jnp.where(lane_iota == cur, bprow, 0),
                       axis=0, keepdims=True)
        path_ref[pl.ds(j - 1, 1), :] = prev
        return prev

    jax.lax.fori_loop(0, S - 1, back, last)


@functools.partial(jax.jit, static_argnames=())
def kernel(x, y, A):
    xt = jnp.transpose(x, (1, 2, 0))
    yt = jnp.transpose(y, (1, 0))
    AT = jnp.transpose(A, (1, 0))

    path_t, nll = pl.pallas_call(
        _crf_kernel,
        out_shape=(
            jax.ShapeDtypeStruct((S, B), jnp.int32),
            jax.ShapeDtypeStruct((1, 1), jnp.float32),
        ),
        scratch_shapes=[
            pltpu.VMEM((S, L, B), jnp.int32),
            pltpu.VMEM((L, L, B), jnp.float32),
        ],
    )(xt, yt, A, AT)

    return path_t.T, nll[0, 0]
